# baseline (speedup 1.0000x reference)
import functools

import jax
import jax.numpy as jnp
from jax.experimental import pallas as pl
from jax.experimental.pallas import tpu as pltpu

_E = 8
_I = 128
_SI = 256
_TB = 1024


def _moe_half(xb, wab_ref, wc_ref, wr_ref):
    xhi = xb.astype(jnp.bfloat16)
    ncols = _SI + _E * _I
    logits = jnp.dot(xb, wr_ref[...], preferred_element_type=jnp.float32)

    idx = jax.lax.broadcasted_iota(jnp.int32, logits.shape, 1)
    m1 = jnp.max(logits, axis=-1, keepdims=True)
    i1 = jnp.min(jnp.where(logits == m1, idx, _E), axis=-1, keepdims=True)
    mask1 = idx == i1
    lm = jnp.where(mask1, -jnp.inf, logits)
    m2 = jnp.max(lm, axis=-1, keepdims=True)
    i2 = jnp.min(jnp.where(lm == m2, idx, _E), axis=-1, keepdims=True)
    mask2 = idx == i2
    d = jnp.exp(m2 - m1)
    w1 = 1.0 / (1.0 + d)
    w2 = 1.0 - w1

    gu = jnp.dot(xhi, wab_ref[...], preferred_element_type=jnp.float32)
    g = gu[:, :ncols]
    u = gu[:, ncols:]
    h = (g * jax.nn.sigmoid(g)) * u

    wcols = [jnp.ones((h.shape[0], _SI), jnp.float32)]
    for e in range(_E):
        we = jnp.where(mask1[:, e:e + 1], w1, 0.0) + \
             jnp.where(mask2[:, e:e + 1], w2, 0.0)
        wcols.append(jnp.broadcast_to(we, (h.shape[0], _I)))
    wexp = jnp.concatenate(wcols, axis=1)

    hw = (h * wexp).astype(jnp.bfloat16)
    return jnp.dot(hw, wc_ref[...], preferred_element_type=jnp.float32)


def _moe_body(x_ref, wab_ref, wc_ref, wr_ref, out_ref):
    out_ref[...] = _moe_half(x_ref[...], wab_ref, wc_ref, wr_ref)


@functools.partial(jax.jit, static_argnames=())
def kernel(x, Ws1, Ws2, Ws3, W1, W2, W3, Wr):
    B, T, C = x.shape
    ntok = B * T
    x_flat = x.reshape(ntok, C)
    bf = jnp.bfloat16
    wab = jnp.concatenate([
        Ws1.astype(bf), W1.transpose(1, 0, 2).reshape(C, _E * _I).astype(bf),
        Ws2.astype(bf), W2.transpose(1, 0, 2).reshape(C, _E * _I).astype(bf),
    ], axis=1)
    wc = jnp.concatenate(
        [Ws3.astype(bf), W3.reshape(_E * _I, C).astype(bf)], axis=0)
    ncols = _SI + _E * _I
    grid = (ntok // _TB,)
    out = pl.pallas_call(
        _moe_body,
        grid=grid,
        in_specs=[
            pl.BlockSpec((_TB, C), lambda i: (i, 0)),
            pl.BlockSpec((C, 2 * ncols), lambda i: (0, 0)),
            pl.BlockSpec((ncols, C), lambda i: (0, 0)),
            pl.BlockSpec((C, _E), lambda i: (0, 0)),
        ],
        out_specs=pl.BlockSpec((_TB, C), lambda i: (i, 0)),
        out_shape=jax.ShapeDtypeStruct((ntok, C), jnp.float32),
        compiler_params=pltpu.CompilerParams(
            dimension_semantics=("parallel",),
        ),
    )(x_flat, wab, wc, Wr)
    return out.reshape(B, T, C)


# speedup vs baseline: 1.0025x; 1.0025x over previous
import functools

import jax
import jax.numpy as jnp
from jax.experimental import pallas as pl
from jax.experimental.pallas import tpu as pltpu

_E = 8
_I = 128
_SI = 256
_TB = 1024


def _moe_half(xb, wab_ref, wc_ref, wr_ref):
    xhi = xb.astype(jnp.bfloat16)
    ncols = _SI + _E * _I
    logits = jnp.dot(xb, wr_ref[...], preferred_element_type=jnp.float32)

    idx = jax.lax.broadcasted_iota(jnp.int32, logits.shape, 1)
    m1 = jnp.max(logits, axis=-1, keepdims=True)
    i1 = jnp.min(jnp.where(logits == m1, idx, _E), axis=-1, keepdims=True)
    mask1 = idx == i1
    lm = jnp.where(mask1, -jnp.inf, logits)
    m2 = jnp.max(lm, axis=-1, keepdims=True)
    i2 = jnp.min(jnp.where(lm == m2, idx, _E), axis=-1, keepdims=True)
    mask2 = idx == i2
    d = jnp.exp(m2 - m1)
    w1 = 1.0 / (1.0 + d)
    w2 = 1.0 - w1

    gu = jnp.dot(xhi, wab_ref[...], preferred_element_type=jnp.float32)
    g = gu[:, :ncols]
    u = gu[:, ncols:]
    h = (g * jax.nn.sigmoid(g)) * u

    wcols = [jnp.ones((h.shape[0], _SI), jnp.float32)]
    for e in range(_E):
        we = jnp.where(mask1[:, e:e + 1], w1, 0.0) + \
             jnp.where(mask2[:, e:e + 1], w2, 0.0)
        wcols.append(jnp.broadcast_to(we, (h.shape[0], _I)))
    wexp = jnp.concatenate(wcols, axis=1)

    hw = (h * wexp).astype(jnp.bfloat16)
    return jnp.dot(hw, wc_ref[...], preferred_element_type=jnp.float32)


def _moe_body(x_ref, wab_ref, wc_ref, wr_ref, out_ref):
    out_ref[...] = _moe_half(x_ref[...], wab_ref, wc_ref, wr_ref)


@functools.partial(jax.jit, static_argnames=())
def kernel(x, Ws1, Ws2, Ws3, W1, W2, W3, Wr):
    B, T, C = x.shape
    ntok = B * T
    x_flat = x.reshape(ntok, C)
    bf = jnp.bfloat16
    wab = jnp.concatenate([
        Ws1.astype(bf), W1.astype(bf).transpose(1, 0, 2).reshape(C, _E * _I),
        Ws2.astype(bf), W2.astype(bf).transpose(1, 0, 2).reshape(C, _E * _I),
    ], axis=1)
    wc = jnp.concatenate(
        [Ws3.astype(bf), W3.reshape(_E * _I, C).astype(bf)], axis=0)
    ncols = _SI + _E * _I
    grid = (ntok // _TB,)
    out = pl.pallas_call(
        _moe_body,
        grid=grid,
        in_specs=[
            pl.BlockSpec((_TB, C), lambda i: (i, 0)),
            pl.BlockSpec((C, 2 * ncols), lambda i: (0, 0)),
            pl.BlockSpec((ncols, C), lambda i: (0, 0)),
            pl.BlockSpec((C, _E), lambda i: (0, 0)),
        ],
        out_specs=pl.BlockSpec((_TB, C), lambda i: (i, 0)),
        out_shape=jax.ShapeDtypeStruct((ntok, C), jnp.float32),
        compiler_params=pltpu.CompilerParams(
            dimension_semantics=("parallel",),
        ),
    )(x_flat, wab, wc, Wr)
    return out.reshape(B, T, C)
